# baseline (speedup 1.0000x reference)
import functools

import jax
import jax.numpy as jnp
from jax import lax
from jax.experimental import pallas as pl
from jax.experimental.pallas import tpu as pltpu
from jax.experimental.pallas import tpu_sc as plsc

IN_SIZE = 4096
D = 2048

_info = plsc.get_sparse_core_info()
NC, NS = _info.num_cores, _info.num_subcores
NW = NC * NS
B_PER_W = IN_SIZE // NW
CHUNK = 8
NCHUNKS = B_PER_W // CHUNK

_mesh = plsc.VectorSubcoreMesh(core_axis_name="c", subcore_axis_name="s")


NBUF = 7
LOOKAHEAD = 3


@functools.partial(
    pl.kernel,
    mesh=_mesh,
    out_type=jax.ShapeDtypeStruct((IN_SIZE, D), jnp.float32),
    scratch_types=[
        pltpu.VMEM((B_PER_W,), jnp.int32),
        [pltpu.VMEM((CHUNK, D), jnp.float32) for _ in range(NBUF)],
        [pltpu.SemaphoreType.DMA for _ in range(NBUF)],
        [pltpu.SemaphoreType.DMA for _ in range(NBUF)],
    ],
)
def _permute_sc(x_hbm, perm_hbm, out_hbm, idx_v, bufs, gsems, wsems):
    wid = lax.axis_index("s") * NC + lax.axis_index("c")
    base = wid * B_PER_W
    pltpu.sync_copy(perm_hbm.at[pl.ds(base, B_PER_W)], idx_v)

    def gather(c):
        b = c % NBUF
        return pltpu.async_copy(
            x_hbm.at[idx_v.at[pl.ds(c * CHUNK, CHUNK)]], bufs[b], gsems[b]
        )

    def write(c):
        b = c % NBUF
        return pltpu.async_copy(
            bufs[b], out_hbm.at[pl.ds(base + c * CHUNK, CHUNK)], wsems[b]
        )

    gh = {}
    wh = {}
    for c in range(NCHUNKS + LOOKAHEAD):
        if c < NCHUNKS:
            if c >= NBUF:
                wh[c - NBUF].wait()
            gh[c] = gather(c)
        cw = c - LOOKAHEAD
        if 0 <= cw < NCHUNKS:
            gh[cw].wait()
            wh[cw] = write(cw)
    for c in range(max(0, NCHUNKS - NBUF), NCHUNKS):
        wh[c].wait()


def kernel(x, y, perm):
    out = _permute_sc(x, perm.astype(jnp.int32))
    return (out, jnp.zeros((), dtype=x.dtype))


# speedup vs baseline: 1.0163x; 1.0163x over previous
import functools

import jax
import jax.numpy as jnp
from jax import lax
from jax.experimental import pallas as pl
from jax.experimental.pallas import tpu as pltpu
from jax.experimental.pallas import tpu_sc as plsc

IN_SIZE = 4096
D = 2048

_info = plsc.get_sparse_core_info()
NC, NS = _info.num_cores, _info.num_subcores
NW = NC * NS
B_PER_W = IN_SIZE // NW
CHUNK = 8
NCHUNKS = B_PER_W // CHUNK
NSLOT = 6
WPAIR = 2
NWRITES = NCHUNKS // WPAIR

_mesh = plsc.VectorSubcoreMesh(core_axis_name="c", subcore_axis_name="s")


@functools.partial(
    pl.kernel,
    mesh=_mesh,
    out_type=jax.ShapeDtypeStruct((IN_SIZE, D), jnp.float32),
    scratch_types=[
        pltpu.VMEM((B_PER_W,), jnp.int32),
        pltpu.VMEM((NSLOT * CHUNK, D), jnp.float32),
        [pltpu.SemaphoreType.DMA for _ in range(NSLOT)],
        [pltpu.SemaphoreType.DMA for _ in range(NSLOT // WPAIR)],
    ],
)
def _permute_sc(x_hbm, perm_hbm, out_hbm, idx_v, ring, gsems, wsems):
    wid = lax.axis_index("s") * NC + lax.axis_index("c")
    base = wid * B_PER_W
    pltpu.sync_copy(perm_hbm.at[pl.ds(base, B_PER_W)], idx_v)

    def gather(c):
        s = c % NSLOT
        return pltpu.async_copy(
            x_hbm.at[idx_v.at[pl.ds(c * CHUNK, CHUNK)]],
            ring.at[pl.ds(s * CHUNK, CHUNK)],
            gsems[s],
        )

    def write(k):
        s = (WPAIR * k) % NSLOT
        return pltpu.async_copy(
            ring.at[pl.ds(s * CHUNK, WPAIR * CHUNK)],
            out_hbm.at[pl.ds(base + k * WPAIR * CHUNK, WPAIR * CHUNK)],
            wsems[s // WPAIR],
        )

    gh = {}
    wh = {}
    for c in range(NCHUNKS):
        if c >= NSLOT and (c - NSLOT) % WPAIR == 0:
            wh[(c - NSLOT) // WPAIR].wait()
        gh[c] = gather(c)
        cw = c - 3
        if cw >= 1 and cw % WPAIR == 1:
            k = (cw - 1) // WPAIR
            gh[cw - 1].wait()
            gh[cw].wait()
            wh[k] = write(k)
    for k in range(NWRITES):
        if k not in wh:
            gh[WPAIR * k].wait()
            gh[WPAIR * k + 1].wait()
            wh[k] = write(k)
    for k in range(NWRITES - NSLOT // WPAIR, NWRITES):
        wh[k].wait()


def kernel(x, y, perm):
    out = _permute_sc(x, perm.astype(jnp.int32))
    return (out, jnp.zeros((), dtype=x.dtype))
